# Initial kernel scaffold; baseline (speedup 1.0000x reference)
import jax
import jax.numpy as jnp
from jax.experimental import pallas as pl


def kernel(x, pe):
    raise NotImplementedError("write your pallas kernel here")


# speedup vs baseline: 1.7144x; 1.7144x over previous
import jax
import jax.numpy as jnp
from jax import lax
from jax.experimental import pallas as pl
from jax.experimental.pallas import tpu as pltpu
from jax.experimental.pallas import tpu_sc as plsc

_B, _S = 16384, 100
_N = _B * _S
_V, _D = 100, 256
_NC, _NS = 2, 16
_NW = _NC * _NS
_ROWS_PER_W = _N // _NW
_CHUNK = 128
_CHUNKS = _ROWS_PER_W // _CHUNK
_LANES = 16

_MAGIC = 8388608.0


def _make_sc_gather():
    mesh = plsc.VectorSubcoreMesh(core_axis_name="c", subcore_axis_name="s")

    @pl.kernel(
        out_type=jax.ShapeDtypeStruct((_N, _D), jnp.float32),
        mesh=mesh,
        scratch_types=[
            pltpu.VMEM((_CHUNK,), jnp.float32),
            pltpu.VMEM((_CHUNK,), jnp.int32),
            pltpu.VMEM((_CHUNK, _D), jnp.float32),
            pltpu.SemaphoreType.DMA,
        ],
    )
    def sc_gather(x_hbm, pe_hbm, out_hbm, x_v, idx_v, rows_v, sem):
        cid = lax.axis_index("c")
        sid = lax.axis_index("s")
        wid = cid * _NS + sid

        @pl.loop(0, _CHUNKS)
        def _(c):
            base = wid * _ROWS_PER_W + c * _CHUNK
            pltpu.sync_copy(x_hbm.at[pl.ds(base, _CHUNK)], x_v)
            for i in range(_CHUNK // _LANES):
                sl = pl.ds(i * _LANES, _LANES)
                y = jnp.maximum(x_v[sl], 0.01) * 100.0
                r = (y + _MAGIC) - _MAGIC
                idx_v[sl] = r.astype(jnp.int32) - 1
            pltpu.async_copy(pe_hbm.at[idx_v], rows_v, sem).wait()
            pltpu.sync_copy(rows_v, out_hbm.at[pl.ds(base, _CHUNK)])

    return sc_gather


_sc_gather = _make_sc_gather()


def kernel(x, pe):
    out = _sc_gather(x.reshape(_N), pe)
    return out.reshape(_B, _S, _D)


# speedup vs baseline: 2.6948x; 1.5718x over previous
import jax
import jax.numpy as jnp
from jax import lax
from jax.experimental import pallas as pl
from jax.experimental.pallas import tpu as pltpu
from jax.experimental.pallas import tpu_sc as plsc

_B, _S = 16384, 100
_N = _B * _S
_V, _D = 100, 256
_NC, _NS = 2, 16
_NW = _NC * _NS
_ROWS_PER_W = _N // _NW
_CHUNK = 128
_CHUNKS = _ROWS_PER_W // _CHUNK
_PAIRS = _CHUNKS // 2
_LANES = 16
_MAGIC = 8388608.0


def _make_sc_gather():
    mesh = plsc.VectorSubcoreMesh(core_axis_name="c", subcore_axis_name="s")

    @pl.kernel(
        out_type=jax.ShapeDtypeStruct((_N, _D), jnp.float32),
        mesh=mesh,
        scratch_types=[
            pltpu.VMEM((_CHUNK,), jnp.float32),
            pltpu.VMEM((_CHUNK,), jnp.int32),
            pltpu.VMEM((_CHUNK, _D), jnp.float32),
            pltpu.VMEM((_CHUNK, _D), jnp.float32),
            pltpu.SemaphoreType.DMA,
            pltpu.SemaphoreType.DMA,
            pltpu.SemaphoreType.DMA,
            pltpu.SemaphoreType.DMA,
        ],
    )
    def sc_gather(x_hbm, rep_hbm, out_hbm, x_v, idx_v, rows_a, rows_b,
                  sga, sgb, swa, swb):
        cid = lax.axis_index("c")
        sid = lax.axis_index("s")
        wid = cid * _NS + sid
        woff = wid * _ROWS_PER_W

        def load_idx(c):
            pltpu.sync_copy(x_hbm.at[pl.ds(woff + c * _CHUNK, _CHUNK)], x_v)
            for i in range(_CHUNK // _LANES):
                sl = pl.ds(i * _LANES, _LANES)
                y = jnp.maximum(x_v[sl], 0.01) * 100.0
                r = (y + _MAGIC) - _MAGIC
                idx_v[sl] = r.astype(jnp.int32) + (wid * _V - 1)

        def out_slice(c):
            return out_hbm.at[pl.ds(woff + c * _CHUNK, _CHUNK)]

        @pl.loop(0, _PAIRS)
        def _(p):
            c0 = p * 2

            load_idx(c0)

            @pl.when(p > 0)
            def _():
                pltpu.make_async_copy(rows_a, out_slice(c0), swa).wait()

            ga = pltpu.async_copy(rep_hbm.at[idx_v], rows_a, sga)

            load_idx(c0 + 1)

            @pl.when(p > 0)
            def _():
                pltpu.make_async_copy(rows_b, out_slice(c0 + 1), swb).wait()

            gb = pltpu.async_copy(rep_hbm.at[idx_v], rows_b, sgb)

            ga.wait()
            pltpu.async_copy(rows_a, out_slice(c0), swa)
            gb.wait()
            pltpu.async_copy(rows_b, out_slice(c0 + 1), swb)

        pltpu.make_async_copy(rows_a, out_slice(_CHUNKS - 2), swa).wait()
        pltpu.make_async_copy(rows_b, out_slice(_CHUNKS - 1), swb).wait()

    return sc_gather


_sc_gather = _make_sc_gather()


def kernel(x, pe):
    pe_rep = jnp.tile(pe, (_NW, 1))
    out = _sc_gather(x.reshape(_N), pe_rep)
    return out.reshape(_B, _S, _D)


# speedup vs baseline: 8.2302x; 3.0541x over previous
import jax
import jax.numpy as jnp
from jax import lax
from jax.experimental import pallas as pl
from jax.experimental.pallas import tpu as pltpu
from jax.experimental.pallas import tpu_sc as plsc

_B, _S = 16384, 100
_N = _B * _S
_V, _D = 100, 256
_NC, _NS = 2, 16
_NW = _NC * _NS
_ROWS_PER_W = _N // _NW
_CHUNK = 128
_CHUNKS = _ROWS_PER_W // _CHUNK
_PAIRS = _CHUNKS // 2
_LANES = 16
_MAGIC = 8388608.0


def _make_sc_gather():
    mesh = plsc.VectorSubcoreMesh(core_axis_name="c", subcore_axis_name="s")

    @pl.kernel(
        out_type=jax.ShapeDtypeStruct((_N, _D), jnp.float32),
        mesh=mesh,
        scratch_types=[
            pltpu.VMEM((_CHUNK,), jnp.float32),
            pltpu.VMEM((_CHUNK,), jnp.int32),
            pltpu.VMEM((_CHUNK, _D), jnp.float32),
            pltpu.VMEM((_CHUNK, _D), jnp.float32),
            pltpu.SemaphoreType.DMA,
            pltpu.SemaphoreType.DMA,
            pltpu.SemaphoreType.DMA,
            pltpu.SemaphoreType.DMA,
        ],
    )
    def sc_gather(x_hbm, rep_hbm, out_hbm, x_v, idx_v, rows_a, rows_b,
                  sga, sgb, swa, swb):
        cid = lax.axis_index("c")
        sid = lax.axis_index("s")
        wid = cid * _NS + sid
        woff = wid * _ROWS_PER_W

        def load_idx(c):
            pltpu.sync_copy(x_hbm.at[pl.ds(woff + c * _CHUNK, _CHUNK)], x_v)
            for i in range(_CHUNK // _LANES):
                sl = pl.ds(i * _LANES, _LANES)
                y = jnp.maximum(x_v[sl], 0.01) * 100.0
                r = (y + _MAGIC) - _MAGIC
                idx_v[sl] = r.astype(jnp.int32) + (wid * _V - 1)

        def out_slice(c):
            return out_hbm.at[pl.ds(woff + c * _CHUNK, _CHUNK)]

        @pl.loop(0, _PAIRS)
        def _(p):
            c0 = p * 2

            load_idx(c0)

            @pl.when(p > 0)
            def _():
                pltpu.make_async_copy(rows_a, out_slice(c0), swa).wait()

            ga = pltpu.async_copy(rep_hbm.at[idx_v], rows_a, sga)

            load_idx(c0 + 1)

            @pl.when(p > 0)
            def _():
                pltpu.make_async_copy(rows_b, out_slice(c0 + 1), swb).wait()

            gb = pltpu.async_copy(rep_hbm.at[idx_v], rows_b, sgb)

            ga.wait()
            pltpu.async_copy(rows_a, out_slice(c0), swa)
            gb.wait()
            pltpu.async_copy(rows_b, out_slice(c0 + 1), swb)

        pltpu.make_async_copy(rows_a, out_slice(_CHUNKS - 2), swa).wait()
        pltpu.make_async_copy(rows_b, out_slice(_CHUNKS - 1), swb).wait()

    return sc_gather


_sc_gather = _make_sc_gather()


def kernel(x, pe):
    pe_rep = jnp.tile(pe, (_NW, 1))
    xt = x.T.reshape(_N)
    out = _sc_gather(xt, pe_rep)
    return out.reshape(_S, _B, _D).transpose(1, 0, 2)


# speedup vs baseline: 8.3632x; 1.0162x over previous
import jax
import jax.numpy as jnp
from jax import lax
from jax.experimental import pallas as pl
from jax.experimental.pallas import tpu as pltpu
from jax.experimental.pallas import tpu_sc as plsc

_B, _S = 16384, 100
_N = _B * _S
_V, _D = 100, 256
_NC, _NS = 2, 16
_NW = _NC * _NS
_ROWS_PER_W = _N // _NW
_CHUNK = 128
_CHUNKS = _ROWS_PER_W // _CHUNK
_PAIRS = _CHUNKS // 2
_LANES = 16
_MAGIC = 8388608.0


def _make_sc_gather():
    mesh = plsc.VectorSubcoreMesh(core_axis_name="c", subcore_axis_name="s")

    @pl.kernel(
        out_type=jax.ShapeDtypeStruct((_N, _D), jnp.float32),
        mesh=mesh,
        scratch_types=[
            pltpu.VMEM((_CHUNK,), jnp.float32),
            pltpu.VMEM((_CHUNK,), jnp.int32),
            pltpu.VMEM((_CHUNK, _D), jnp.float32),
            pltpu.VMEM((_CHUNK, _D), jnp.float32),
            pltpu.VMEM((_CHUNK, _D), jnp.float32),
            pltpu.SemaphoreType.DMA,
            pltpu.SemaphoreType.DMA,
            pltpu.SemaphoreType.DMA,
            pltpu.SemaphoreType.DMA,
            pltpu.SemaphoreType.DMA,
            pltpu.SemaphoreType.DMA,
        ],
    )
    def sc_gather(x_hbm, rep_hbm, out_hbm, x_v, idx_v, rows_a, rows_b, rows_c,
                  sga, sgb, sgc, swa, swb, swc):
        cid = lax.axis_index("c")
        sid = lax.axis_index("s")
        wid = cid * _NS + sid
        woff = wid * _ROWS_PER_W

        def load_idx(c):
            pltpu.sync_copy(x_hbm.at[pl.ds(woff + c * _CHUNK, _CHUNK)], x_v)
            for i in range(_CHUNK // _LANES):
                sl = pl.ds(i * _LANES, _LANES)
                y = jnp.maximum(x_v[sl], 0.01) * 100.0
                r = (y + _MAGIC) - _MAGIC
                idx_v[sl] = r.astype(jnp.int32) + (wid * _V - 1)

        def out_slice(c):
            return out_hbm.at[pl.ds(woff + c * _CHUNK, _CHUNK)]

        bufs = (rows_a, rows_b, rows_c)
        gsems = (sga, sgb, sgc)
        wsems = (swa, swb, swc)

        def start_gather(c, rows, sg):
            load_idx(c)
            pltpu.async_copy(rep_hbm.at[idx_v], rows, sg)

        def wait_gather(rows, sg):
            pltpu.make_async_copy(rep_hbm.at[idx_v], rows, sg).wait()

        def start_write(c, rows, sw):
            pltpu.async_copy(rows, out_slice(c), sw)

        def wait_write(c, rows, sw):
            pltpu.make_async_copy(rows, out_slice(c), sw).wait()

        start_gather(0, rows_a, sga)
        start_gather(1, rows_b, sgb)
        wait_gather(rows_a, sga)
        start_write(0, rows_a, swa)
        start_gather(2, rows_c, sgc)
        wait_gather(rows_b, sgb)
        start_write(1, rows_b, swb)

        @pl.loop(0, (_CHUNKS - 4) // 3)
        def _(p):
            c = 3 + p * 3
            for k in range(3):
                rows, sg, sw = bufs[k], gsems[k], wsems[k]
                prows, psg, psw = bufs[(k - 1) % 3], gsems[(k - 1) % 3], wsems[(k - 1) % 3]
                wait_write(c + k - 3, rows, sw)
                start_gather(c + k, rows, sg)
                wait_gather(prows, psg)
                start_write(c + k - 1, prows, psw)

        wait_write(_CHUNKS - 4, rows_a, swa)
        start_gather(_CHUNKS - 1, rows_a, sga)
        wait_gather(rows_c, sgc)
        start_write(_CHUNKS - 2, rows_c, swc)
        wait_gather(rows_a, sga)
        start_write(_CHUNKS - 1, rows_a, swa)
        wait_write(_CHUNKS - 3, rows_b, swb)
        wait_write(_CHUNKS - 2, rows_c, swc)
        wait_write(_CHUNKS - 1, rows_a, swa)

    return sc_gather


_sc_gather = _make_sc_gather()


def kernel(x, pe):
    pe_rep = jnp.tile(pe, (_NW, 1))
    xt = x.T.reshape(_N)
    out = _sc_gather(xt, pe_rep)
    return out.reshape(_S, _B, _D).transpose(1, 0, 2)
